# baseline (speedup 1.0000x reference)
import jax
import jax.numpy as jnp
from jax.experimental import pallas as pl
from jax.experimental.pallas import tpu as pltpu


def _add_row_body(tf_id_ref, table_ref, x_ref, o_ref):
    del tf_id_ref
    o_ref[...] = x_ref[...] + table_ref[0]


def kernel(x, tf_table, tf_id):
    B, S, D = x.shape
    R = B * S
    xf = x.reshape(R, D)
    tbl3 = tf_table.reshape(tf_table.shape[0], 1, D)
    tf_id_arr = jnp.asarray(tf_id, dtype=jnp.int32).reshape(1)

    BLK = 512
    grid = (R // BLK,)
    out = pl.pallas_call(
        _add_row_body,
        grid_spec=pltpu.PrefetchScalarGridSpec(
            num_scalar_prefetch=1,
            grid=grid,
            in_specs=[
                pl.BlockSpec((1, 1, D), lambda i, tf_id_ref: (tf_id_ref[0], 0, 0)),
                pl.BlockSpec((BLK, D), lambda i, tf_id_ref: (i, 0)),
            ],
            out_specs=pl.BlockSpec((BLK, D), lambda i, tf_id_ref: (i, 0)),
        ),
        out_shape=jax.ShapeDtypeStruct((R, D), x.dtype),
    )(tf_id_arr, tbl3, xf)
    return out.reshape(B, S, D)


# speedup vs baseline: 1.1077x; 1.1077x over previous
import jax
import jax.numpy as jnp
from jax.experimental import pallas as pl
from jax.experimental.pallas import tpu as pltpu


def _add_row_body(tf_id_ref, table_ref, x_ref, o_ref):
    del tf_id_ref
    o_ref[...] = x_ref[...] + table_ref[0]


def kernel(x, tf_table, tf_id):
    B, S, D = x.shape
    R = B * S
    xf = x.reshape(R, D)
    tbl3 = tf_table.reshape(tf_table.shape[0], 1, D)
    tf_id_arr = jnp.asarray(tf_id, dtype=jnp.int32).reshape(1)

    BLK = 2048
    grid = (R // BLK,)
    out = pl.pallas_call(
        _add_row_body,
        grid_spec=pltpu.PrefetchScalarGridSpec(
            num_scalar_prefetch=1,
            grid=grid,
            in_specs=[
                pl.BlockSpec((1, 1, D), lambda i, tf_id_ref: (tf_id_ref[0], 0, 0)),
                pl.BlockSpec((BLK, D), lambda i, tf_id_ref: (i, 0)),
            ],
            out_specs=pl.BlockSpec((BLK, D), lambda i, tf_id_ref: (i, 0)),
        ),
        out_shape=jax.ShapeDtypeStruct((R, D), x.dtype),
        compiler_params=pltpu.CompilerParams(
            dimension_semantics=("arbitrary",),
        ),
    )(tf_id_arr, tbl3, xf)
    return out.reshape(B, S, D)
